# Initial kernel scaffold; baseline (speedup 1.0000x reference)
import jax
import jax.numpy as jnp
from jax.experimental import pallas as pl


def kernel(nfeats, efeats, edge_index, W_msg1, b_msg1, W_apply1, b_apply1, W_msg2, b_msg2, W_apply2, b_apply2):
    raise NotImplementedError("write your pallas kernel here")


# speedup vs baseline: 2.6983x; 2.6983x over previous
import functools

import jax
import jax.numpy as jnp
from jax import lax
from jax.experimental import pallas as pl
from jax.experimental.pallas import tpu as pltpu
from jax.experimental.pallas import tpu_sc as plsc

N_NODES = 10000
N_PAD = 10240
E_EDGES = 320000
E_PAD = 327680
NW = 32
CHUNKS = 80
CHUNK = 128
GROUPS = 10
GCH = 8
ROWS_PER_TILE = N_PAD // 16
EW = 128


def _mesh():
    return plsc.VectorSubcoreMesh(core_axis_name="c", subcore_axis_name="s")


def _sc_scatter_h():
    out_type = jax.ShapeDtypeStruct((2, N_PAD, 128), jnp.float32)
    scratch = [
        pltpu.VMEM((GCH, CHUNK), jnp.int32),
        pltpu.VMEM((GCH, CHUNK), jnp.int32),
        pltpu.VMEM((CHUNK, 128), jnp.float32),
        pltpu.VMEM((CHUNK, 128), jnp.float32),
        pltpu.VMEM_SHARED((N_PAD, 128), jnp.float32),
        pltpu.SemaphoreType.DMA,
        pltpu.SemaphoreType.DMA,
    ]

    def body(h_hbm, src_hbm, dst_hbm, z128, out_h,
             src_v, dst_v, rows_a, rows_b, acc_h, sem_a, sem_b):
        c = lax.axis_index("c")
        s = lax.axis_index("s")
        wid = c * 16 + s
        r0 = s * ROWS_PER_TILE

        pltpu.sync_copy(z128.at[pl.ds(r0, ROWS_PER_TILE)],
                        acc_h.at[pl.ds(r0, ROWS_PER_TILE)])
        plsc.subcore_barrier()

        def group(g, carry):
            pltpu.sync_copy(src_hbm.at[wid, g], src_v)
            pltpu.sync_copy(dst_hbm.at[wid, g], dst_v)
            bufs = (rows_a, rows_b)
            sems = (sem_a, sem_b)
            pltpu.async_copy(h_hbm.at[src_v.at[0]], rows_a, sem_a)
            for k in range(GCH):
                cur, scur = bufs[k % 2], sems[k % 2]
                pltpu.make_async_copy(h_hbm.at[src_v.at[k]], cur, scur).wait()
                if k + 1 < GCH:
                    nxt, snxt = bufs[(k + 1) % 2], sems[(k + 1) % 2]
                    pltpu.async_copy(h_hbm.at[src_v.at[k + 1]], nxt, snxt)
                pltpu.sync_copy(cur, acc_h.at[dst_v.at[k]], add=True)
            return carry

        lax.fori_loop(0, GROUPS, group, None)

        plsc.subcore_barrier()
        pltpu.sync_copy(acc_h.at[pl.ds(r0, ROWS_PER_TILE)],
                        out_h.at[c, pl.ds(r0, ROWS_PER_TILE)])

    return pl.kernel(body, mesh=_mesh(), out_type=out_type, scratch_types=scratch)


def _dense_body(h_b, sa, sb, ea, eb, wmt, wme, bm, wat, wab, ba, o_ref):
    f32 = jnp.float32
    Sh = sa[...] + sb[...]
    Se = ea[...] + eb[...]
    deg = Se[:, 16:17]
    summ = (jnp.dot(Sh, wmt[...], preferred_element_type=f32)
            + jnp.dot(Se, wme[...], preferred_element_type=f32))
    invd = 1.0 / jnp.maximum(deg, 1.0)
    h_neigh = summ * invd + bm[...] * (deg > 0).astype(f32)
    o = (jnp.dot(h_b[...], wat[...], preferred_element_type=f32)
         + jnp.dot(h_neigh, wab[...], preferred_element_type=f32)
         + ba[...])
    o_ref[...] = jnp.maximum(o, 0.0)


def _dense_layer(h_pad, sh_parts, se_parts, W_msg, b_msg, W_apply, b_apply):
    blk = 1024
    grid = (N_PAD // blk,)
    row_spec = lambda w: pl.BlockSpec((blk, w), lambda i: (i, 0))
    full = lambda a: pl.BlockSpec(a.shape, lambda i: (0,) * a.ndim)
    wmt = W_msg[:128]
    wme = jnp.concatenate([W_msg[128:144], jnp.zeros((EW - 16, 128), jnp.float32)], 0)
    bm = b_msg.reshape(1, 128)
    wat = W_apply[:128]
    wab = W_apply[128:256]
    ba = b_apply.reshape(1, 128)
    return pl.pallas_call(
        _dense_body,
        grid=grid,
        in_specs=[row_spec(128), row_spec(128), row_spec(128),
                  row_spec(EW), row_spec(EW),
                  full(wmt), full(wme), full(bm), full(wat), full(wab), full(ba)],
        out_specs=row_spec(128),
        out_shape=jax.ShapeDtypeStruct((N_PAD, 128), jnp.float32),
    )(h_pad, sh_parts[0], sh_parts[1], se_parts[0], se_parts[1],
      wmt, wme, bm, wat, wab, ba)


def kernel(nfeats, efeats, edge_index, W_msg1, b_msg1, W_apply1, b_apply1,
           W_msg2, b_msg2, W_apply2, b_apply2):
    src = edge_index[0].astype(jnp.int32)
    dst = edge_index[1].astype(jnp.int32)
    pad = E_PAD - E_EDGES
    src_p = jnp.concatenate([src, jnp.zeros((pad,), jnp.int32)]).reshape(
        NW, GROUPS, GCH, CHUNK)
    dst_pad_rows = (N_NODES + (jnp.arange(pad, dtype=jnp.int32) % (N_PAD - N_NODES)))
    dst_p = jnp.concatenate([dst, dst_pad_rows]).reshape(NW, GROUPS, GCH, CHUNK)

    ef = efeats.reshape(E_EDGES, 16)
    ef_plus = jnp.concatenate(
        [ef, jnp.ones((E_EDGES, 1), jnp.float32),
         jnp.zeros((E_EDGES, EW - 17), jnp.float32)], 1)
    ef_plus = jnp.concatenate([ef_plus, jnp.zeros((pad, EW), jnp.float32)], 0)

    h0 = jnp.zeros((N_PAD, 128), jnp.float32).at[:N_NODES].set(nfeats.reshape(N_NODES, 128))
    z128 = jnp.zeros((N_PAD, 128), jnp.float32)
    eidx_p = jnp.arange(E_PAD, dtype=jnp.int32).reshape(NW, GROUPS, GCH, CHUNK)

    sc_h = _sc_scatter_h()
    sh1 = sc_h(h0, src_p, dst_p, z128)
    ef_dep, _ = lax.optimization_barrier((ef_plus, sh1))
    se = sc_h(ef_dep, eidx_p, dst_p, z128)
    h1 = _dense_layer(h0, sh1, se, W_msg1, b_msg1, W_apply1, b_apply1)
    sh2 = sc_h(h1, src_p, dst_p, z128)
    h2 = _dense_layer(h1, sh2, se, W_msg2, b_msg2, W_apply2, b_apply2)
    return h2[:N_NODES]


# speedup vs baseline: 3.0961x; 1.1474x over previous
import functools

import jax
import jax.numpy as jnp
from jax import lax
from jax.experimental import pallas as pl
from jax.experimental.pallas import tpu as pltpu
from jax.experimental.pallas import tpu_sc as plsc

N_NODES = 10000
N_PAD = 10240
E_EDGES = 320000
E_PAD = 327680
NW = 32
CHUNKS = 80
CHUNK = 128
GROUPS = 10
GCH = 8
ROWS_PER_TILE = N_PAD // 16
EW = 128


def _mesh():
    return plsc.VectorSubcoreMesh(core_axis_name="c", subcore_axis_name="s")


def _sc_scatter_h():
    out_type = jax.ShapeDtypeStruct((2, N_PAD, 128), jnp.float32)
    scratch = [
        pltpu.VMEM((GCH, CHUNK), jnp.int32),
        pltpu.VMEM((GCH, CHUNK), jnp.int32),
        pltpu.VMEM((CHUNK, 128), jnp.float32),
        pltpu.VMEM((CHUNK, 128), jnp.float32),
        pltpu.VMEM_SHARED((N_PAD, 128), jnp.float32),
        pltpu.SemaphoreType.DMA,
        pltpu.SemaphoreType.DMA,
    ]

    def body(h_hbm, src_hbm, dst_hbm, z128, out_h,
             src_v, dst_v, rows_a, rows_b, acc_h, sem_a, sem_b):
        c = lax.axis_index("c")
        s = lax.axis_index("s")
        wid = c * 16 + s
        r0 = s * ROWS_PER_TILE

        pltpu.sync_copy(z128.at[pl.ds(r0, ROWS_PER_TILE)],
                        acc_h.at[pl.ds(r0, ROWS_PER_TILE)])
        plsc.subcore_barrier()

        def group(g, carry):
            pltpu.sync_copy(src_hbm.at[wid, g], src_v)
            pltpu.sync_copy(dst_hbm.at[wid, g], dst_v)
            bufs = (rows_a, rows_b)
            sems = (sem_a, sem_b)
            pltpu.async_copy(h_hbm.at[src_v.at[0]], rows_a, sem_a)
            for k in range(GCH):
                cur, scur = bufs[k % 2], sems[k % 2]
                pltpu.make_async_copy(h_hbm.at[src_v.at[k]], cur, scur).wait()
                if k + 1 < GCH:
                    nxt, snxt = bufs[(k + 1) % 2], sems[(k + 1) % 2]
                    pltpu.async_copy(h_hbm.at[src_v.at[k + 1]], nxt, snxt)
                pltpu.sync_copy(cur, acc_h.at[dst_v.at[k]], add=True)
            return carry

        lax.fori_loop(0, GROUPS, group, None)

        plsc.subcore_barrier()
        pltpu.sync_copy(acc_h.at[pl.ds(r0, ROWS_PER_TILE)],
                        out_h.at[c, pl.ds(r0, ROWS_PER_TILE)])

    return pl.kernel(body, mesh=_mesh(), out_type=out_type, scratch_types=scratch)


def _sc_scatter_e():
    out_type = jax.ShapeDtypeStruct((2, N_PAD, 128), jnp.float32)
    scratch = [
        pltpu.VMEM((GCH, CHUNK), jnp.int32),
        pltpu.VMEM((16, 128), jnp.float32),
        pltpu.VMEM((16, 128), jnp.float32),
        pltpu.VMEM((CHUNK, 128), jnp.float32),
        pltpu.VMEM((16,), jnp.float32),
        pltpu.VMEM_SHARED((N_PAD, 128), jnp.float32),
        pltpu.SemaphoreType.DMA,
        pltpu.SemaphoreType.DMA,
    ]

    def body(efp_hbm, dst_hbm, z128, onehot_hbm, out_e,
             dst_v, lin_a, lin_b, pay, oh_v, acc_e, sem_a, sem_b):
        c = lax.axis_index("c")
        s = lax.axis_index("s")
        wid = c * 16 + s
        r0 = s * ROWS_PER_TILE

        pltpu.sync_copy(z128.at[pl.ds(r0, ROWS_PER_TILE)],
                        acc_e.at[pl.ds(r0, ROWS_PER_TILE)])

        pltpu.sync_copy(onehot_hbm, oh_v)
        one0 = oh_v[pl.ds(0, 16)]

        def initrow(e, carry):
            pay[e, pl.ds(16, 16)] = one0
            return carry

        lax.fori_loop(0, CHUNK, initrow, None)
        plsc.subcore_barrier()

        bufs = (lin_a, lin_b)
        sems = (sem_a, sem_b)

        def group(g, carry):
            pltpu.sync_copy(dst_hbm.at[wid, g], dst_v)
            row0 = ((wid * GROUPS + g) * GCH) * 16
            pltpu.async_copy(efp_hbm.at[pl.ds(row0, 16)], lin_a, sem_a)
            for k in range(GCH):
                cur, scur = bufs[k % 2], sems[k % 2]
                pltpu.make_async_copy(
                    efp_hbm.at[pl.ds(row0 + k * 16, 16)], cur, scur).wait()
                if k + 1 < GCH:
                    nxt, snxt = bufs[(k + 1) % 2], sems[(k + 1) % 2]
                    pltpu.async_copy(
                        efp_hbm.at[pl.ds(row0 + (k + 1) * 16, 16)], nxt, snxt)

                def repack(e, carry):
                    pay[e, pl.ds(0, 16)] = cur[e // 8, pl.ds((e % 8) * 16, 16)]
                    return carry

                lax.fori_loop(0, CHUNK, repack, None)
                pltpu.sync_copy(pay, acc_e.at[dst_v.at[k]], add=True)
            return carry

        lax.fori_loop(0, GROUPS, group, None)

        plsc.subcore_barrier()
        pltpu.sync_copy(acc_e.at[pl.ds(r0, ROWS_PER_TILE)],
                        out_e.at[c, pl.ds(r0, ROWS_PER_TILE)])

    return pl.kernel(body, mesh=_mesh(), out_type=out_type, scratch_types=scratch)


def _dense_body(h_b, sa, sb, ea, eb, wmt, wme, bm, wat, wab, ba, o_ref):
    f32 = jnp.float32
    Sh = sa[...] + sb[...]
    Se = ea[:, :16] + eb[:, :16]
    deg = ea[:, 16:17] + eb[:, 16:17]
    summ = (jnp.dot(Sh, wmt[...], preferred_element_type=f32)
            + jnp.dot(Se, wme[...], preferred_element_type=f32))
    invd = 1.0 / jnp.maximum(deg, 1.0)
    h_neigh = summ * invd + bm[...] * (deg > 0).astype(f32)
    o = (jnp.dot(h_b[...], wat[...], preferred_element_type=f32)
         + jnp.dot(h_neigh, wab[...], preferred_element_type=f32)
         + ba[...])
    o_ref[...] = jnp.maximum(o, 0.0)


def _dense_layer(h_pad, sh_parts, se_parts, W_msg, b_msg, W_apply, b_apply):
    blk = 1024
    grid = (N_PAD // blk,)
    row_spec = lambda w: pl.BlockSpec((blk, w), lambda i: (i, 0))
    full = lambda a: pl.BlockSpec(a.shape, lambda i: (0,) * a.ndim)
    wmt = W_msg[:128]
    wme = W_msg[128:144]
    bm = b_msg.reshape(1, 128)
    wat = W_apply[:128]
    wab = W_apply[128:256]
    ba = b_apply.reshape(1, 128)
    return pl.pallas_call(
        _dense_body,
        grid=grid,
        in_specs=[row_spec(128), row_spec(128), row_spec(128),
                  row_spec(EW), row_spec(EW),
                  full(wmt), full(wme), full(bm), full(wat), full(wab), full(ba)],
        out_specs=row_spec(128),
        out_shape=jax.ShapeDtypeStruct((N_PAD, 128), jnp.float32),
    )(h_pad, sh_parts[0], sh_parts[1], se_parts[0], se_parts[1],
      wmt, wme, bm, wat, wab, ba)


def kernel(nfeats, efeats, edge_index, W_msg1, b_msg1, W_apply1, b_apply1,
           W_msg2, b_msg2, W_apply2, b_apply2):
    src = edge_index[0].astype(jnp.int32)
    dst = edge_index[1].astype(jnp.int32)
    pad = E_PAD - E_EDGES
    src_p = jnp.concatenate([src, jnp.zeros((pad,), jnp.int32)]).reshape(
        NW, GROUPS, GCH, CHUNK)
    dst_pad_rows = (N_NODES + (jnp.arange(pad, dtype=jnp.int32) % (N_PAD - N_NODES)))
    dst_p = jnp.concatenate([dst, dst_pad_rows]).reshape(NW, GROUPS, GCH, CHUNK)

    efp = jnp.concatenate(
        [efeats.reshape(E_EDGES * 16 // 128, 128),
         jnp.zeros((pad * 16 // 128, 128), jnp.float32)], 0)

    h0 = jnp.zeros((N_PAD, 128), jnp.float32).at[:N_NODES].set(nfeats.reshape(N_NODES, 128))
    z128 = jnp.zeros((N_PAD, 128), jnp.float32)

    sc_h = _sc_scatter_h()
    sh1 = sc_h(h0, src_p, dst_p, z128)
    efp_dep, _ = lax.optimization_barrier((efp, sh1))
    onehot = jnp.zeros((16,), jnp.float32).at[0].set(1.0)
    se = _sc_scatter_e()(efp_dep, dst_p, z128, onehot)
    h1 = _dense_layer(h0, sh1, se, W_msg1, b_msg1, W_apply1, b_apply1)
    sh2 = sc_h(h1, src_p, dst_p, z128)
    h2 = _dense_layer(h1, sh2, se, W_msg2, b_msg2, W_apply2, b_apply2)
    return h2[:N_NODES]


# speedup vs baseline: 7.3157x; 2.3629x over previous
import functools

import jax
import jax.numpy as jnp
from jax import lax
from jax.experimental import pallas as pl
from jax.experimental.pallas import tpu as pltpu
from jax.experimental.pallas import tpu_sc as plsc

N_NODES = 10000
N_PAD = 10240
E_EDGES = 320000
E_PAD = 327680
NW = 32
CHUNKS = 80
CHUNK = 128
GROUPS = 10
GCH = 8
ROWS_PER_TILE = N_PAD // 16
EW = 128


def _mesh():
    return plsc.VectorSubcoreMesh(core_axis_name="c", subcore_axis_name="s")


def _sc_scatter_h():
    out_type = jax.ShapeDtypeStruct((2, N_PAD, 128), jnp.float32)
    scratch = [
        pltpu.VMEM((GCH, CHUNK), jnp.int32),
        pltpu.VMEM((GCH, CHUNK), jnp.int32),
        pltpu.VMEM((CHUNK, 128), jnp.float32),
        pltpu.VMEM((CHUNK, 128), jnp.float32),
        pltpu.VMEM_SHARED((N_PAD, 128), jnp.float32),
        pltpu.SemaphoreType.DMA,
        pltpu.SemaphoreType.DMA,
    ]

    def body(h_hbm, src_hbm, dst_hbm, z128, out_h,
             src_v, dst_v, rows_a, rows_b, acc_h, sem_a, sem_b):
        c = lax.axis_index("c")
        s = lax.axis_index("s")
        wid = c * 16 + s
        r0 = s * ROWS_PER_TILE

        pltpu.sync_copy(z128.at[pl.ds(r0, ROWS_PER_TILE)],
                        acc_h.at[pl.ds(r0, ROWS_PER_TILE)])
        plsc.subcore_barrier()

        def group(g, carry):
            pltpu.sync_copy(src_hbm.at[wid, g], src_v)
            pltpu.sync_copy(dst_hbm.at[wid, g], dst_v)
            bufs = (rows_a, rows_b)
            sems = (sem_a, sem_b)
            pltpu.async_copy(h_hbm.at[src_v.at[0]], rows_a, sem_a)
            for k in range(GCH):
                cur, scur = bufs[k % 2], sems[k % 2]
                pltpu.make_async_copy(h_hbm.at[src_v.at[k]], cur, scur).wait()
                if k + 1 < GCH:
                    nxt, snxt = bufs[(k + 1) % 2], sems[(k + 1) % 2]
                    pltpu.async_copy(h_hbm.at[src_v.at[k + 1]], nxt, snxt)
                pltpu.sync_copy(cur, acc_h.at[dst_v.at[k]], add=True)
            return carry

        lax.fori_loop(0, GROUPS, group, None)

        plsc.subcore_barrier()
        pltpu.sync_copy(acc_h.at[pl.ds(r0, ROWS_PER_TILE)],
                        out_h.at[c, pl.ds(r0, ROWS_PER_TILE)])

    return pl.kernel(body, mesh=_mesh(), out_type=out_type, scratch_types=scratch)


def _sc_scatter_e():
    out_type = jax.ShapeDtypeStruct((2, N_PAD, 128), jnp.float32)
    scratch = [
        pltpu.VMEM((GCH, CHUNK), jnp.int32),
        pltpu.VMEM((16, 128), jnp.float32),
        pltpu.VMEM((16, 128), jnp.float32),
        pltpu.VMEM((CHUNK, 128), jnp.float32),
        pltpu.VMEM((16,), jnp.float32),
        pltpu.VMEM_SHARED((N_PAD, 128), jnp.float32),
        pltpu.SemaphoreType.DMA,
        pltpu.SemaphoreType.DMA,
    ]

    def body(efp_hbm, dst_hbm, z128, onehot_hbm, out_e,
             dst_v, lin_a, lin_b, pay, oh_v, acc_e, sem_a, sem_b):
        c = lax.axis_index("c")
        s = lax.axis_index("s")
        wid = c * 16 + s
        r0 = s * ROWS_PER_TILE

        pltpu.sync_copy(z128.at[pl.ds(r0, ROWS_PER_TILE)],
                        acc_e.at[pl.ds(r0, ROWS_PER_TILE)])

        pltpu.sync_copy(onehot_hbm, oh_v)
        one0 = oh_v[pl.ds(0, 16)]

        def initrow(e, carry):
            pay[e, pl.ds(16, 16)] = one0
            return carry

        lax.fori_loop(0, CHUNK, initrow, None)
        plsc.subcore_barrier()

        bufs = (lin_a, lin_b)
        sems = (sem_a, sem_b)

        def group(g, carry):
            pltpu.sync_copy(dst_hbm.at[wid, g], dst_v)
            row0 = ((wid * GROUPS + g) * GCH) * 16
            pltpu.async_copy(efp_hbm.at[pl.ds(row0, 16)], lin_a, sem_a)
            for k in range(GCH):
                cur, scur = bufs[k % 2], sems[k % 2]
                pltpu.make_async_copy(
                    efp_hbm.at[pl.ds(row0 + k * 16, 16)], cur, scur).wait()
                if k + 1 < GCH:
                    nxt, snxt = bufs[(k + 1) % 2], sems[(k + 1) % 2]
                    pltpu.async_copy(
                        efp_hbm.at[pl.ds(row0 + (k + 1) * 16, 16)], nxt, snxt)

                def repack(e, carry):
                    pay[e, pl.ds(0, 16)] = cur[e // 8, pl.ds((e % 8) * 16, 16)]
                    return carry

                lax.fori_loop(0, CHUNK, repack, None)
                pltpu.sync_copy(pay, acc_e.at[dst_v.at[k]], add=True)
            return carry

        lax.fori_loop(0, GROUPS, group, None)

        plsc.subcore_barrier()
        pltpu.sync_copy(acc_e.at[pl.ds(r0, ROWS_PER_TILE)],
                        out_e.at[c, pl.ds(r0, ROWS_PER_TILE)])

    return pl.kernel(body, mesh=_mesh(), out_type=out_type, scratch_types=scratch)


def _dense_body(h_b, sa, sb, ea, eb, wmt, wme, bm, wat, wab, ba, o_ref):
    f32 = jnp.float32
    Sh = sa[...] + sb[...]
    Se = ea[:, :16] + eb[:, :16]
    deg = ea[:, 16:17] + eb[:, 16:17]
    summ = (jnp.dot(Sh, wmt[...], preferred_element_type=f32)
            + jnp.dot(Se, wme[...], preferred_element_type=f32))
    invd = 1.0 / jnp.maximum(deg, 1.0)
    h_neigh = summ * invd + bm[...] * (deg > 0).astype(f32)
    o = (jnp.dot(h_b[...], wat[...], preferred_element_type=f32)
         + jnp.dot(h_neigh, wab[...], preferred_element_type=f32)
         + ba[...])
    o_ref[...] = jnp.maximum(o, 0.0)


def _dense_layer(h_pad, sh_parts, se_parts, W_msg, b_msg, W_apply, b_apply):
    blk = 1024
    grid = (N_PAD // blk,)
    row_spec = lambda w: pl.BlockSpec((blk, w), lambda i: (i, 0))
    full = lambda a: pl.BlockSpec(a.shape, lambda i: (0,) * a.ndim)
    wmt = W_msg[:128]
    wme = W_msg[128:144]
    bm = b_msg.reshape(1, 128)
    wat = W_apply[:128]
    wab = W_apply[128:256]
    ba = b_apply.reshape(1, 128)
    return pl.pallas_call(
        _dense_body,
        grid=grid,
        in_specs=[row_spec(128), row_spec(128), row_spec(128),
                  row_spec(EW), row_spec(EW),
                  full(wmt), full(wme), full(bm), full(wat), full(wab), full(ba)],
        out_specs=row_spec(128),
        out_shape=jax.ShapeDtypeStruct((N_PAD, 128), jnp.float32),
    )(h_pad, sh_parts[0], sh_parts[1], se_parts[0], se_parts[1],
      wmt, wme, bm, wat, wab, ba)


def kernel(nfeats, efeats, edge_index, W_msg1, b_msg1, W_apply1, b_apply1,
           W_msg2, b_msg2, W_apply2, b_apply2):
    src = edge_index[0].astype(jnp.int32)
    dst = edge_index[1].astype(jnp.int32)
    pad = E_PAD - E_EDGES
    src_pad_rows = jnp.arange(pad, dtype=jnp.int32) % N_NODES
    src_p = jnp.concatenate([src, src_pad_rows]).reshape(
        NW, GROUPS, GCH, CHUNK)
    dst_pad_rows = (N_NODES + (jnp.arange(pad, dtype=jnp.int32) % (N_PAD - N_NODES)))
    dst_p = jnp.concatenate([dst, dst_pad_rows]).reshape(NW, GROUPS, GCH, CHUNK)

    efp = jnp.concatenate(
        [efeats.reshape(E_EDGES * 16 // 128, 128),
         jnp.zeros((pad * 16 // 128, 128), jnp.float32)], 0)

    h0 = jnp.zeros((N_PAD, 128), jnp.float32).at[:N_NODES].set(nfeats.reshape(N_NODES, 128))
    z128 = jnp.zeros((N_PAD, 128), jnp.float32)

    sc_h = _sc_scatter_h()
    sh1 = sc_h(h0, src_p, dst_p, z128)
    efp_dep, _ = lax.optimization_barrier((efp, sh1))
    onehot = jnp.zeros((16,), jnp.float32).at[0].set(1.0)
    se = _sc_scatter_e()(efp_dep, dst_p, z128, onehot)
    h1 = _dense_layer(h0, sh1, se, W_msg1, b_msg1, W_apply1, b_apply1)
    sh2 = sc_h(h1, src_p, dst_p, z128)
    h2 = _dense_layer(h1, sh2, se, W_msg2, b_msg2, W_apply2, b_apply2)
    return h2[:N_NODES]


# speedup vs baseline: 7.9238x; 1.0831x over previous
import functools

import jax
import jax.numpy as jnp
from jax import lax
from jax.experimental import pallas as pl
from jax.experimental.pallas import tpu as pltpu
from jax.experimental.pallas import tpu_sc as plsc

N_NODES = 10000
N_PAD = 10240
E_EDGES = 320000
E_PAD = 327680
NW = 32
CHUNKS = 80
CHUNK = 128
GROUPS = 10
GCH = 8
ROWS_PER_TILE = N_PAD // 16
EW = 128


def _mesh():
    return plsc.VectorSubcoreMesh(core_axis_name="c", subcore_axis_name="s")


def _sc_scatter_h():
    out_type = jax.ShapeDtypeStruct((2, N_PAD, 128), jnp.float32)
    scratch = [
        pltpu.VMEM((GCH, CHUNK), jnp.int32),
        pltpu.VMEM((GCH, CHUNK), jnp.int32),
        pltpu.VMEM((CHUNK, 128), jnp.float32),
        pltpu.VMEM((CHUNK, 128), jnp.float32),
        pltpu.VMEM_SHARED((N_PAD, 128), jnp.float32),
        pltpu.SemaphoreType.DMA,
        pltpu.SemaphoreType.DMA,
    ]

    def body(h_hbm, src_hbm, dst_hbm, z128, out_h,
             src_v, dst_v, rows_a, rows_b, acc_h, sem_a, sem_b):
        c = lax.axis_index("c")
        s = lax.axis_index("s")
        wid = c * 16 + s
        r0 = s * ROWS_PER_TILE

        pltpu.sync_copy(z128.at[pl.ds(r0, ROWS_PER_TILE)],
                        acc_h.at[pl.ds(r0, ROWS_PER_TILE)])
        plsc.subcore_barrier()

        def group(g, carry):
            pltpu.sync_copy(src_hbm.at[wid, g], src_v)
            pltpu.sync_copy(dst_hbm.at[wid, g], dst_v)
            bufs = (rows_a, rows_b)
            sems = (sem_a, sem_b)
            pltpu.async_copy(h_hbm.at[src_v.at[0]], rows_a, sem_a)
            for k in range(GCH):
                cur, scur = bufs[k % 2], sems[k % 2]
                pltpu.make_async_copy(h_hbm.at[src_v.at[k]], cur, scur).wait()
                if k + 1 < GCH:
                    nxt, snxt = bufs[(k + 1) % 2], sems[(k + 1) % 2]
                    pltpu.async_copy(h_hbm.at[src_v.at[k + 1]], nxt, snxt)
                pltpu.sync_copy(cur, acc_h.at[dst_v.at[k]], add=True)
            return carry

        lax.fori_loop(0, GROUPS, group, None)

        plsc.subcore_barrier()
        pltpu.sync_copy(acc_h.at[pl.ds(r0, ROWS_PER_TILE)],
                        out_h.at[c, pl.ds(r0, ROWS_PER_TILE)])

    return pl.kernel(body, mesh=_mesh(), out_type=out_type, scratch_types=scratch)


def _sc_scatter_e():
    out_type = jax.ShapeDtypeStruct((2, N_PAD, 128), jnp.float32)
    scratch = [
        pltpu.VMEM((GCH, CHUNK), jnp.int32),
        pltpu.VMEM((16, 128), jnp.float32),
        pltpu.VMEM((16, 128), jnp.float32),
        pltpu.VMEM((CHUNK, 128), jnp.float32),
        pltpu.VMEM((CHUNK, 128), jnp.float32),
        pltpu.VMEM((16,), jnp.float32),
        pltpu.VMEM_SHARED((N_PAD, 128), jnp.float32),
        pltpu.SemaphoreType.DMA,
        pltpu.SemaphoreType.DMA,
        pltpu.SemaphoreType.DMA,
        pltpu.SemaphoreType.DMA,
    ]

    def body(efp_hbm, dst_hbm, z128, onehot_hbm, out_e,
             dst_v, lin_a, lin_b, pay_a, pay_b, oh_v, acc_e,
             sem_a, sem_b, ssem_a, ssem_b):
        c = lax.axis_index("c")
        s = lax.axis_index("s")
        wid = c * 16 + s
        r0 = s * ROWS_PER_TILE

        pltpu.sync_copy(z128.at[pl.ds(r0, ROWS_PER_TILE)],
                        acc_e.at[pl.ds(r0, ROWS_PER_TILE)])

        pltpu.sync_copy(onehot_hbm, oh_v)
        one0 = oh_v[pl.ds(0, 16)]

        def initrow(e, carry):
            pay_a[e, pl.ds(16, 16)] = one0
            pay_b[e, pl.ds(16, 16)] = one0
            return carry

        lax.fori_loop(0, CHUNK, initrow, None)
        plsc.subcore_barrier()

        bufs = (lin_a, lin_b)
        sems = (sem_a, sem_b)
        pays = (pay_a, pay_b)
        ssems = (ssem_a, ssem_b)

        def group(g, carry):
            @pl.when(g > 0)
            def _():
                pltpu.make_async_copy(
                    pay_a, acc_e.at[dst_v.at[GCH - 2]], ssem_a).wait()
                pltpu.make_async_copy(
                    pay_b, acc_e.at[dst_v.at[GCH - 1]], ssem_b).wait()

            pltpu.sync_copy(dst_hbm.at[wid, g], dst_v)
            row0 = ((wid * GROUPS + g) * GCH) * 16
            pltpu.async_copy(efp_hbm.at[pl.ds(row0, 16)], lin_a, sem_a)
            for k in range(GCH):
                cur, scur = bufs[k % 2], sems[k % 2]
                pay, ssem = pays[k % 2], ssems[k % 2]
                pltpu.make_async_copy(
                    efp_hbm.at[pl.ds(row0 + k * 16, 16)], cur, scur).wait()
                if k + 1 < GCH:
                    nxt, snxt = bufs[(k + 1) % 2], sems[(k + 1) % 2]
                    pltpu.async_copy(
                        efp_hbm.at[pl.ds(row0 + (k + 1) * 16, 16)], nxt, snxt)
                if k >= 2:
                    pltpu.make_async_copy(
                        pay, acc_e.at[dst_v.at[k - 2]], ssem).wait()

                def repack(e, carry):
                    pay[e, pl.ds(0, 16)] = cur[e // 8, pl.ds((e % 8) * 16, 16)]
                    return carry

                lax.fori_loop(0, CHUNK, repack, None)
                pltpu.async_copy(pay, acc_e.at[dst_v.at[k]], ssem, add=True)
            return carry

        lax.fori_loop(0, GROUPS, group, None)

        pltpu.make_async_copy(pay_a, acc_e.at[dst_v.at[GCH - 2]], ssem_a).wait()
        pltpu.make_async_copy(pay_b, acc_e.at[dst_v.at[GCH - 1]], ssem_b).wait()
        plsc.subcore_barrier()
        pltpu.sync_copy(acc_e.at[pl.ds(r0, ROWS_PER_TILE)],
                        out_e.at[c, pl.ds(r0, ROWS_PER_TILE)])

    return pl.kernel(body, mesh=_mesh(), out_type=out_type, scratch_types=scratch)


def _dense_body(h_b, sa, sb, ea, eb, wmt, wme, bm, wat, wab, ba, o_ref):
    f32 = jnp.float32
    Sh = sa[0] + sb[0]
    Se = ea[0, :, :16] + eb[0, :, :16]
    deg = ea[0, :, 16:17] + eb[0, :, 16:17]
    summ = (jnp.dot(Sh, wmt[...], preferred_element_type=f32)
            + jnp.dot(Se, wme[...], preferred_element_type=f32))
    invd = 1.0 / jnp.maximum(deg, 1.0)
    h_neigh = summ * invd + bm[...] * (deg > 0).astype(f32)
    o = (jnp.dot(h_b[...], wat[...], preferred_element_type=f32)
         + jnp.dot(h_neigh, wab[...], preferred_element_type=f32)
         + ba[...])
    o_ref[...] = jnp.maximum(o, 0.0)


def _dense_layer(h, sh_parts, se_parts, W_msg, b_msg, W_apply, b_apply):
    blk = 400
    grid = (N_NODES // blk,)
    row_spec = pl.BlockSpec((blk, 128), lambda i: (i, 0))
    part_spec = lambda p: pl.BlockSpec((1, blk, 128), lambda i, _p=p: (_p, i, 0))
    full = lambda a: pl.BlockSpec(a.shape, lambda i: (0,) * a.ndim)
    wmt = W_msg[:128]
    wme = W_msg[128:144]
    bm = b_msg.reshape(1, 128)
    wat = W_apply[:128]
    wab = W_apply[128:256]
    ba = b_apply.reshape(1, 128)
    return pl.pallas_call(
        _dense_body,
        grid=grid,
        in_specs=[row_spec, part_spec(0), part_spec(1),
                  part_spec(0), part_spec(1),
                  full(wmt), full(wme), full(bm), full(wat), full(wab), full(ba)],
        out_specs=row_spec,
        out_shape=jax.ShapeDtypeStruct((N_NODES, 128), jnp.float32),
    )(h, sh_parts, sh_parts, se_parts, se_parts,
      wmt, wme, bm, wat, wab, ba)


def kernel(nfeats, efeats, edge_index, W_msg1, b_msg1, W_apply1, b_apply1,
           W_msg2, b_msg2, W_apply2, b_apply2):
    src = edge_index[0].astype(jnp.int32)
    dst = edge_index[1].astype(jnp.int32)
    pad = E_PAD - E_EDGES
    src_pad_rows = jnp.arange(pad, dtype=jnp.int32) % N_NODES
    src_p = jnp.concatenate([src, src_pad_rows]).reshape(
        NW, GROUPS, GCH, CHUNK)
    dst_pad_rows = (N_NODES + (jnp.arange(pad, dtype=jnp.int32) % (N_PAD - N_NODES)))
    dst_p = jnp.concatenate([dst, dst_pad_rows]).reshape(NW, GROUPS, GCH, CHUNK)

    efp = jnp.concatenate(
        [efeats.reshape(E_EDGES * 16 // 128, 128),
         jnp.zeros((pad * 16 // 128, 128), jnp.float32)], 0)

    h0 = nfeats.reshape(N_NODES, 128)
    z128 = jnp.zeros((N_PAD, 128), jnp.float32)

    sc_h = _sc_scatter_h()
    sh1 = sc_h(h0, src_p, dst_p, z128)
    efp_dep, _ = lax.optimization_barrier((efp, sh1))
    onehot = jnp.zeros((16,), jnp.float32).at[0].set(1.0)
    se = _sc_scatter_e()(efp_dep, dst_p, z128, onehot)
    h1 = _dense_layer(h0, sh1, se, W_msg1, b_msg1, W_apply1, b_apply1)
    sh2 = sc_h(h1, src_p, dst_p, z128)
    h2 = _dense_layer(h1, sh2, se, W_msg2, b_msg2, W_apply2, b_apply2)
    return h2


# speedup vs baseline: 8.2246x; 1.0380x over previous
import functools

import jax
import jax.numpy as jnp
from jax import lax
from jax.experimental import pallas as pl
from jax.experimental.pallas import tpu as pltpu
from jax.experimental.pallas import tpu_sc as plsc

N_NODES = 10000
N_PAD = 10240
E_EDGES = 320000
E_PAD = 327680
NW = 32
CHUNKS = 80
CHUNK = 128
GROUPS = 10
GCH = 8
ROWS_PER_TILE = N_PAD // 16
EW = 128


def _mesh():
    return plsc.VectorSubcoreMesh(core_axis_name="c", subcore_axis_name="s")


def _sc_scatter_h():
    out_type = jax.ShapeDtypeStruct((2, N_PAD, 128), jnp.float32)
    scratch = [
        pltpu.VMEM((GCH, CHUNK), jnp.int32),
        pltpu.VMEM((GCH, CHUNK), jnp.int32),
        pltpu.VMEM((GCH, CHUNK), jnp.int32),
        pltpu.VMEM((GCH, CHUNK), jnp.int32),
        pltpu.VMEM((CHUNK, 128), jnp.float32),
        pltpu.VMEM((CHUNK, 128), jnp.float32),
        pltpu.VMEM_SHARED((N_PAD, 128), jnp.float32),
        pltpu.SemaphoreType.DMA,
        pltpu.SemaphoreType.DMA,
        pltpu.SemaphoreType.DMA,
    ]

    def body(h_hbm, src_hbm, dst_hbm, z128, out_h,
             src_a, dst_a, src_b, dst_b, rows_a, rows_b, acc_h,
             sem_a, sem_b, sem_i):
        c = lax.axis_index("c")
        s = lax.axis_index("s")
        wid = c * 16 + s
        r0 = s * ROWS_PER_TILE

        pltpu.sync_copy(z128.at[pl.ds(r0, ROWS_PER_TILE)],
                        acc_h.at[pl.ds(r0, ROWS_PER_TILE)])
        plsc.subcore_barrier()

        bufs = (rows_a, rows_b)
        sems = (sem_a, sem_b)

        def do_group(src_v, dst_v):
            pltpu.async_copy(h_hbm.at[src_v.at[0]], rows_a, sem_a)
            for k in range(GCH):
                cur, scur = bufs[k % 2], sems[k % 2]
                pltpu.make_async_copy(h_hbm.at[src_v.at[k]], cur, scur).wait()
                if k + 1 < GCH:
                    nxt, snxt = bufs[(k + 1) % 2], sems[(k + 1) % 2]
                    pltpu.async_copy(h_hbm.at[src_v.at[k + 1]], nxt, snxt)
                pltpu.sync_copy(cur, acc_h.at[dst_v.at[k]], add=True)

        def fetch_idx(g, src_v, dst_v):
            pltpu.async_copy(src_hbm.at[wid, g], src_v, sem_i)
            pltpu.async_copy(dst_hbm.at[wid, g], dst_v, sem_i)

        def wait_idx(g, src_v, dst_v):
            pltpu.make_async_copy(src_hbm.at[wid, g], src_v, sem_i).wait()
            pltpu.make_async_copy(dst_hbm.at[wid, g], dst_v, sem_i).wait()

        fetch_idx(0, src_a, dst_a)

        def gpair(g2, carry):
            g = g2 * 2
            wait_idx(g, src_a, dst_a)
            fetch_idx(g + 1, src_b, dst_b)
            do_group(src_a, dst_a)
            wait_idx(g + 1, src_b, dst_b)

            @pl.when(g2 + 1 < GROUPS // 2)
            def _():
                fetch_idx(g + 2, src_a, dst_a)

            do_group(src_b, dst_b)
            return carry

        lax.fori_loop(0, GROUPS // 2, gpair, None)

        plsc.subcore_barrier()
        pltpu.sync_copy(acc_h.at[pl.ds(r0, ROWS_PER_TILE)],
                        out_h.at[c, pl.ds(r0, ROWS_PER_TILE)])

    return pl.kernel(body, mesh=_mesh(), out_type=out_type, scratch_types=scratch)


def _sc_scatter_e():
    out_type = jax.ShapeDtypeStruct((2, N_PAD, 128), jnp.float32)
    scratch = [
        pltpu.VMEM((GCH, CHUNK), jnp.int32),
        pltpu.VMEM((16, 128), jnp.float32),
        pltpu.VMEM((16, 128), jnp.float32),
        pltpu.VMEM((CHUNK, 128), jnp.float32),
        pltpu.VMEM((CHUNK, 128), jnp.float32),
        pltpu.VMEM((16,), jnp.float32),
        pltpu.VMEM_SHARED((N_PAD, 128), jnp.float32),
        pltpu.SemaphoreType.DMA,
        pltpu.SemaphoreType.DMA,
        pltpu.SemaphoreType.DMA,
        pltpu.SemaphoreType.DMA,
    ]

    def body(efp_hbm, dst_hbm, z128, onehot_hbm, out_e,
             dst_v, lin_a, lin_b, pay_a, pay_b, oh_v, acc_e,
             sem_a, sem_b, ssem_a, ssem_b):
        c = lax.axis_index("c")
        s = lax.axis_index("s")
        wid = c * 16 + s
        r0 = s * ROWS_PER_TILE

        pltpu.sync_copy(z128.at[pl.ds(r0, ROWS_PER_TILE)],
                        acc_e.at[pl.ds(r0, ROWS_PER_TILE)])

        pltpu.sync_copy(onehot_hbm, oh_v)
        one0 = oh_v[pl.ds(0, 16)]

        def initrow(e, carry):
            pay_a[e, pl.ds(16, 16)] = one0
            pay_b[e, pl.ds(16, 16)] = one0
            return carry

        lax.fori_loop(0, CHUNK, initrow, None)
        plsc.subcore_barrier()

        bufs = (lin_a, lin_b)
        sems = (sem_a, sem_b)
        pays = (pay_a, pay_b)
        ssems = (ssem_a, ssem_b)

        def group(g, carry):
            @pl.when(g > 0)
            def _():
                pltpu.make_async_copy(
                    pay_a, acc_e.at[dst_v.at[GCH - 2]], ssem_a).wait()
                pltpu.make_async_copy(
                    pay_b, acc_e.at[dst_v.at[GCH - 1]], ssem_b).wait()

            pltpu.sync_copy(dst_hbm.at[wid, g], dst_v)
            row0 = ((wid * GROUPS + g) * GCH) * 16
            pltpu.async_copy(efp_hbm.at[pl.ds(row0, 16)], lin_a, sem_a)
            for k in range(GCH):
                cur, scur = bufs[k % 2], sems[k % 2]
                pay, ssem = pays[k % 2], ssems[k % 2]
                pltpu.make_async_copy(
                    efp_hbm.at[pl.ds(row0 + k * 16, 16)], cur, scur).wait()
                if k + 1 < GCH:
                    nxt, snxt = bufs[(k + 1) % 2], sems[(k + 1) % 2]
                    pltpu.async_copy(
                        efp_hbm.at[pl.ds(row0 + (k + 1) * 16, 16)], nxt, snxt)
                if k >= 2:
                    pltpu.make_async_copy(
                        pay, acc_e.at[dst_v.at[k - 2]], ssem).wait()

                def repack(e, carry):
                    pay[e, pl.ds(0, 16)] = cur[e // 8, pl.ds((e % 8) * 16, 16)]
                    return carry

                lax.fori_loop(0, CHUNK, repack, None)
                pltpu.async_copy(pay, acc_e.at[dst_v.at[k]], ssem, add=True)
            return carry

        lax.fori_loop(0, GROUPS, group, None)

        pltpu.make_async_copy(pay_a, acc_e.at[dst_v.at[GCH - 2]], ssem_a).wait()
        pltpu.make_async_copy(pay_b, acc_e.at[dst_v.at[GCH - 1]], ssem_b).wait()
        plsc.subcore_barrier()
        pltpu.sync_copy(acc_e.at[pl.ds(r0, ROWS_PER_TILE)],
                        out_e.at[c, pl.ds(r0, ROWS_PER_TILE)])

    return pl.kernel(body, mesh=_mesh(), out_type=out_type, scratch_types=scratch)


def _dense_body(h_b, sa, sb, ea, eb, wmt, wme, bm, wat, wab, ba, o_ref):
    f32 = jnp.float32
    Sh = sa[0] + sb[0]
    Se = ea[0, :, :16] + eb[0, :, :16]
    deg = ea[0, :, 16:17] + eb[0, :, 16:17]
    summ = (jnp.dot(Sh, wmt[...], preferred_element_type=f32)
            + jnp.dot(Se, wme[...], preferred_element_type=f32))
    invd = 1.0 / jnp.maximum(deg, 1.0)
    h_neigh = summ * invd + bm[...] * (deg > 0).astype(f32)
    o = (jnp.dot(h_b[...], wat[...], preferred_element_type=f32)
         + jnp.dot(h_neigh, wab[...], preferred_element_type=f32)
         + ba[...])
    o_ref[...] = jnp.maximum(o, 0.0)


def _dense_layer(h, sh_parts, se_parts, W_msg, b_msg, W_apply, b_apply):
    blk = 400
    grid = (N_NODES // blk,)
    row_spec = pl.BlockSpec((blk, 128), lambda i: (i, 0))
    part_spec = lambda p: pl.BlockSpec((1, blk, 128), lambda i, _p=p: (_p, i, 0))
    full = lambda a: pl.BlockSpec(a.shape, lambda i: (0,) * a.ndim)
    wmt = W_msg[:128]
    wme = W_msg[128:144]
    bm = b_msg.reshape(1, 128)
    wat = W_apply[:128]
    wab = W_apply[128:256]
    ba = b_apply.reshape(1, 128)
    return pl.pallas_call(
        _dense_body,
        grid=grid,
        in_specs=[row_spec, part_spec(0), part_spec(1),
                  part_spec(0), part_spec(1),
                  full(wmt), full(wme), full(bm), full(wat), full(wab), full(ba)],
        out_specs=row_spec,
        out_shape=jax.ShapeDtypeStruct((N_NODES, 128), jnp.float32),
    )(h, sh_parts, sh_parts, se_parts, se_parts,
      wmt, wme, bm, wat, wab, ba)


def kernel(nfeats, efeats, edge_index, W_msg1, b_msg1, W_apply1, b_apply1,
           W_msg2, b_msg2, W_apply2, b_apply2):
    src = edge_index[0].astype(jnp.int32)
    dst = edge_index[1].astype(jnp.int32)
    pad = E_PAD - E_EDGES
    src_pad_rows = jnp.arange(pad, dtype=jnp.int32) % N_NODES
    src_p = jnp.concatenate([src, src_pad_rows]).reshape(
        NW, GROUPS, GCH, CHUNK)
    dst_pad_rows = (N_NODES + (jnp.arange(pad, dtype=jnp.int32) % (N_PAD - N_NODES)))
    dst_p = jnp.concatenate([dst, dst_pad_rows]).reshape(NW, GROUPS, GCH, CHUNK)

    efp = jnp.concatenate(
        [efeats.reshape(E_EDGES * 16 // 128, 128),
         jnp.zeros((pad * 16 // 128, 128), jnp.float32)], 0)

    h0 = nfeats.reshape(N_NODES, 128)
    z128 = jnp.zeros((N_PAD, 128), jnp.float32)

    sc_h = _sc_scatter_h()
    sh1 = sc_h(h0, src_p, dst_p, z128)
    efp_dep, _ = lax.optimization_barrier((efp, sh1))
    onehot = jnp.zeros((16,), jnp.float32).at[0].set(1.0)
    se = _sc_scatter_e()(efp_dep, dst_p, z128, onehot)
    h1 = _dense_layer(h0, sh1, se, W_msg1, b_msg1, W_apply1, b_apply1)
    sh2 = sc_h(h1, src_p, dst_p, z128)
    h2 = _dense_layer(h1, sh2, se, W_msg2, b_msg2, W_apply2, b_apply2)
    return h2


# speedup vs baseline: 8.2618x; 1.0045x over previous
import jax
import jax.numpy as jnp
from jax import lax
from jax.experimental import pallas as pl
from jax.experimental.pallas import tpu as pltpu
from jax.experimental.pallas import tpu_sc as plsc

N_NODES = 10000
N_PAD = 10240
E_EDGES = 320000
E_PAD = 327680
NW = 32
CHUNKS = 80
CHUNK = 128
GROUPS = 10
GCH = 8
ROWS_PER_TILE = N_PAD // 16


def _mesh():
    return plsc.VectorSubcoreMesh(core_axis_name="c", subcore_axis_name="s")


def _sc_scatter_h():
    out_type = jax.ShapeDtypeStruct((2, N_PAD, 128), jnp.float32)
    scratch = [
        pltpu.VMEM((GCH, CHUNK), jnp.int32),
        pltpu.VMEM((GCH, CHUNK), jnp.int32),
        pltpu.VMEM((GCH, CHUNK), jnp.int32),
        pltpu.VMEM((GCH, CHUNK), jnp.int32),
        pltpu.VMEM((CHUNK, 128), jnp.float32),
        pltpu.VMEM((CHUNK, 128), jnp.float32),
        pltpu.VMEM_SHARED((N_PAD, 128), jnp.float32),
        pltpu.SemaphoreType.DMA,
        pltpu.SemaphoreType.DMA,
        pltpu.SemaphoreType.DMA,
    ]

    def body(h_hbm, src_hbm, dst_hbm, z128, out_h,
             src_a, dst_a, src_b, dst_b, rows_a, rows_b, acc_h,
             sem_a, sem_b, sem_i):
        c = lax.axis_index("c")
        s = lax.axis_index("s")
        wid = c * 16 + s
        r0 = s * ROWS_PER_TILE

        pltpu.sync_copy(z128.at[pl.ds(r0, ROWS_PER_TILE)],
                        acc_h.at[pl.ds(r0, ROWS_PER_TILE)])
        plsc.subcore_barrier()

        bufs = (rows_a, rows_b)
        sems = (sem_a, sem_b)

        def do_group(src_v, dst_v):
            pltpu.async_copy(h_hbm.at[src_v.at[0]], rows_a, sem_a)
            for k in range(GCH):
                cur, scur = bufs[k % 2], sems[k % 2]
                pltpu.make_async_copy(h_hbm.at[src_v.at[k]], cur, scur).wait()
                if k + 1 < GCH:
                    nxt, snxt = bufs[(k + 1) % 2], sems[(k + 1) % 2]
                    pltpu.async_copy(h_hbm.at[src_v.at[k + 1]], nxt, snxt)
                pltpu.sync_copy(cur, acc_h.at[dst_v.at[k]], add=True)

        def fetch_idx(g, src_v, dst_v):
            pltpu.async_copy(src_hbm.at[wid, g], src_v, sem_i)
            pltpu.async_copy(dst_hbm.at[wid, g], dst_v, sem_i)

        def wait_idx(g, src_v, dst_v):
            pltpu.make_async_copy(src_hbm.at[wid, g], src_v, sem_i).wait()
            pltpu.make_async_copy(dst_hbm.at[wid, g], dst_v, sem_i).wait()

        fetch_idx(0, src_a, dst_a)

        def gpair(g2, carry):
            g = g2 * 2
            wait_idx(g, src_a, dst_a)
            fetch_idx(g + 1, src_b, dst_b)
            do_group(src_a, dst_a)
            wait_idx(g + 1, src_b, dst_b)

            @pl.when(g2 + 1 < GROUPS // 2)
            def _():
                fetch_idx(g + 2, src_a, dst_a)

            do_group(src_b, dst_b)
            return carry

        lax.fori_loop(0, GROUPS // 2, gpair, None)

        plsc.subcore_barrier()
        pltpu.sync_copy(acc_h.at[pl.ds(r0, ROWS_PER_TILE)],
                        out_h.at[c, pl.ds(r0, ROWS_PER_TILE)])

    return pl.kernel(body, mesh=_mesh(), out_type=out_type, scratch_types=scratch)


def _sc_scatter_e():
    out_type = jax.ShapeDtypeStruct((2, N_PAD, 128), jnp.float32)
    scratch = [
        pltpu.VMEM((GCH, CHUNK), jnp.int32),
        pltpu.VMEM((16, 128), jnp.float32),
        pltpu.VMEM((16, 128), jnp.float32),
        pltpu.VMEM((CHUNK, 128), jnp.float32),
        pltpu.VMEM((CHUNK, 128), jnp.float32),
        pltpu.VMEM((16,), jnp.float32),
        pltpu.VMEM_SHARED((N_PAD, 128), jnp.float32),
        pltpu.SemaphoreType.DMA,
        pltpu.SemaphoreType.DMA,
        pltpu.SemaphoreType.DMA,
        pltpu.SemaphoreType.DMA,
    ]

    def body(efp_hbm, dst_hbm, z128, onehot_hbm, out_e,
             dst_v, lin_a, lin_b, pay_a, pay_b, oh_v, acc_e,
             sem_a, sem_b, ssem_a, ssem_b):
        c = lax.axis_index("c")
        s = lax.axis_index("s")
        wid = c * 16 + s
        r0 = s * ROWS_PER_TILE

        pltpu.sync_copy(z128.at[pl.ds(r0, ROWS_PER_TILE)],
                        acc_e.at[pl.ds(r0, ROWS_PER_TILE)])

        pltpu.sync_copy(onehot_hbm, oh_v)
        one0 = oh_v[pl.ds(0, 16)]

        def initrow(e, carry):
            pay_a[e, pl.ds(16, 16)] = one0
            pay_b[e, pl.ds(16, 16)] = one0
            return carry

        lax.fori_loop(0, CHUNK, initrow, None)
        plsc.subcore_barrier()

        bufs = (lin_a, lin_b)
        sems = (sem_a, sem_b)
        pays = (pay_a, pay_b)
        ssems = (ssem_a, ssem_b)

        def group(g, carry):
            @pl.when(g > 0)
            def _():
                pltpu.make_async_copy(
                    pay_a, acc_e.at[dst_v.at[GCH - 2]], ssem_a).wait()
                pltpu.make_async_copy(
                    pay_b, acc_e.at[dst_v.at[GCH - 1]], ssem_b).wait()

            pltpu.sync_copy(dst_hbm.at[wid, g], dst_v)
            row0 = ((wid * GROUPS + g) * GCH) * 16
            pltpu.async_copy(efp_hbm.at[pl.ds(row0, 16)], lin_a, sem_a)
            for k in range(GCH):
                cur, scur = bufs[k % 2], sems[k % 2]
                pay, ssem = pays[k % 2], ssems[k % 2]
                pltpu.make_async_copy(
                    efp_hbm.at[pl.ds(row0 + k * 16, 16)], cur, scur).wait()
                if k + 1 < GCH:
                    nxt, snxt = bufs[(k + 1) % 2], sems[(k + 1) % 2]
                    pltpu.async_copy(
                        efp_hbm.at[pl.ds(row0 + (k + 1) * 16, 16)], nxt, snxt)
                if k >= 2:
                    pltpu.make_async_copy(
                        pay, acc_e.at[dst_v.at[k - 2]], ssem).wait()

                def repack(e, carry):
                    pay[e, pl.ds(0, 16)] = cur[e // 8, pl.ds((e % 8) * 16, 16)]
                    return carry

                lax.fori_loop(0, CHUNK, repack, None)
                pltpu.async_copy(pay, acc_e.at[dst_v.at[k]], ssem, add=True)
            return carry

        lax.fori_loop(0, GROUPS, group, None)

        pltpu.make_async_copy(pay_a, acc_e.at[dst_v.at[GCH - 2]], ssem_a).wait()
        pltpu.make_async_copy(pay_b, acc_e.at[dst_v.at[GCH - 1]], ssem_b).wait()
        plsc.subcore_barrier()
        pltpu.sync_copy(acc_e.at[pl.ds(r0, ROWS_PER_TILE)],
                        out_e.at[c, pl.ds(r0, ROWS_PER_TILE)])

    return pl.kernel(body, mesh=_mesh(), out_type=out_type, scratch_types=scratch)


def _dense_body(h_b, sa, sb, ea, eb, wmt, wme, bm, wat, wab, ba, o_ref):
    f32 = jnp.float32
    Sh = sa[0] + sb[0]
    Se = ea[0, :, :16] + eb[0, :, :16]
    deg = ea[0, :, 16:17] + eb[0, :, 16:17]
    summ = (jnp.dot(Sh, wmt[...], preferred_element_type=f32)
            + jnp.dot(Se, wme[...], preferred_element_type=f32))
    invd = 1.0 / jnp.maximum(deg, 1.0)
    h_neigh = summ * invd + bm[...] * (deg > 0).astype(f32)
    o = (jnp.dot(h_b[...], wat[...], preferred_element_type=f32)
         + jnp.dot(h_neigh, wab[...], preferred_element_type=f32)
         + ba[...])
    o_ref[...] = jnp.maximum(o, 0.0)


def _dense_layer(h, sh_parts, se_parts, W_msg, b_msg, W_apply, b_apply):
    blk = 400
    grid = (N_NODES // blk,)
    row_spec = pl.BlockSpec((blk, 128), lambda i: (i, 0))
    part_spec = lambda p: pl.BlockSpec((1, blk, 128), lambda i, _p=p: (_p, i, 0))
    full = lambda a: pl.BlockSpec(a.shape, lambda i: (0,) * a.ndim)
    wmt = W_msg[:128]
    wme = W_msg[128:144]
    bm = b_msg.reshape(1, 128)
    wat = W_apply[:128]
    wab = W_apply[128:256]
    ba = b_apply.reshape(1, 128)
    return pl.pallas_call(
        _dense_body,
        grid=grid,
        in_specs=[row_spec, part_spec(0), part_spec(1),
                  part_spec(0), part_spec(1),
                  full(wmt), full(wme), full(bm), full(wat), full(wab), full(ba)],
        out_specs=row_spec,
        out_shape=jax.ShapeDtypeStruct((N_NODES, 128), jnp.float32),
    )(h, sh_parts, sh_parts, se_parts, se_parts,
      wmt, wme, bm, wat, wab, ba)


def kernel(nfeats, efeats, edge_index, W_msg1, b_msg1, W_apply1, b_apply1,
           W_msg2, b_msg2, W_apply2, b_apply2):
    src = edge_index[0].astype(jnp.int32)
    dst = edge_index[1].astype(jnp.int32)
    pad = E_PAD - E_EDGES
    src_pad_rows = jnp.arange(pad, dtype=jnp.int32) % N_NODES
    src_p = jnp.concatenate([src, src_pad_rows]).reshape(
        NW, GROUPS, GCH, CHUNK)
    dst_pad_rows = (N_NODES + (jnp.arange(pad, dtype=jnp.int32) % (N_PAD - N_NODES)))
    dst_p = jnp.concatenate([dst, dst_pad_rows]).reshape(NW, GROUPS, GCH, CHUNK)

    efp = jnp.concatenate(
        [efeats.reshape(E_EDGES * 16 // 128, 128),
         jnp.zeros((pad * 16 // 128, 128), jnp.float32)], 0)

    h0 = nfeats.reshape(N_NODES, 128)
    z128 = jnp.zeros((N_PAD, 128), jnp.float32)

    sc_h = _sc_scatter_h()
    sh1 = sc_h(h0, src_p, dst_p, z128)
    efp_dep, _ = lax.optimization_barrier((efp, sh1))
    onehot = jnp.zeros((16,), jnp.float32).at[0].set(1.0)
    se = _sc_scatter_e()(efp_dep, dst_p, z128, onehot)
    h1 = _dense_layer(h0, sh1, se, W_msg1, b_msg1, W_apply1, b_apply1)
    sh2 = sc_h(h1, src_p, dst_p, z128)
    h2 = _dense_layer(h1, sh2, se, W_msg2, b_msg2, W_apply2, b_apply2)
    return h2
